# baseline (speedup 1.0000x reference)
import functools

import jax
import jax.numpy as jnp
from jax import lax
from jax.experimental import pallas as pl
from jax.experimental.pallas import tpu as pltpu
from jax.experimental.pallas import tpu_sc as plsc

B, N, D = 4, 4096, 2048
NS = 1024
NT = N - NS
BLK = 1024
NBLK = NT // BLK
NCAND = 16
WPB = 8
NW = 32
RPW = NS // WPB
RC = 32
NCHUNK = RPW // RC
_BIG = 2**30
_NEG = float("-inf")


def _tc_score_body(x_ref, wd_ref, val_ref, idx_ref):
    nb = pl.program_id(1)
    x = x_ref[0]
    acc = x[:, 0:128] * wd_ref[0:1, :]
    for k in range(1, D // 128):
        acc = acc + x[:, k * 128:(k + 1) * 128] * wd_ref[k:k + 1, :]
    s = jnp.sum(acc, axis=1, keepdims=True)
    m = jnp.max(s)
    ii = lax.broadcasted_iota(jnp.int32, (BLK, 1), 0)
    am = jnp.min(jnp.where(s == m, ii, _BIG))

    @pl.when(nb == 0)
    def _():
        for k in range(NBLK, NCAND):
            val_ref[0, 0, k] = _NEG
            idx_ref[0, 0, k] = _BIG

    val_ref[0, 0, nb] = m
    idx_ref[0, 0, nb] = am + nb * BLK


def _tc_score(emb, wd2):
    return pl.pallas_call(
        _tc_score_body,
        grid=(B, NBLK),
        compiler_params=pltpu.CompilerParams(
            dimension_semantics=("parallel", "arbitrary")),
        in_specs=[
            pl.BlockSpec((1, BLK, D), lambda b, n: (b, n, 0)),
            pl.BlockSpec((D // 128, 128), lambda b, n: (0, 0)),
        ],
        out_specs=[
            pl.BlockSpec((1, 1, NCAND), lambda b, n: (b, 0, 0),
                         memory_space=pltpu.SMEM),
            pl.BlockSpec((1, 1, NCAND), lambda b, n: (b, 0, 0),
                         memory_space=pltpu.SMEM),
        ],
        out_shape=[
            jax.ShapeDtypeStruct((B, 1, NCAND), jnp.float32),
            jax.ShapeDtypeStruct((B, 1, NCAND), jnp.int32),
        ],
    )(emb, wd2)


@functools.lru_cache(maxsize=1)
def _make_sc_score():
    info = plsc.get_sparse_core_info()
    nc = info.num_cores
    mesh = plsc.VectorSubcoreMesh(core_axis_name="c", subcore_axis_name="s")

    @functools.partial(
        pl.kernel,
        mesh=mesh,
        compiler_params=pltpu.CompilerParams(needs_layout_passes=False),
        out_type=(
            jax.ShapeDtypeStruct((NW, 16), jnp.float32),
            jax.ShapeDtypeStruct((NW, 16), jnp.int32),
        ),
        scratch_types=[
            pltpu.VMEM((D,), jnp.float32),
            pltpu.VMEM((RC, D), jnp.float32),
            pltpu.VMEM((16,), jnp.float32),
            pltpu.VMEM((16,), jnp.int32),
            pltpu.SemaphoreType.DMA,
        ],
    )
    def sc_score(emb_hbm, wd_hbm, out_v, out_i, wd_v, buf, bv_v, bi_v, sem):
        wid = lax.axis_index("s") * nc + lax.axis_index("c")
        b = wid // WPB
        c = wid % WPB
        row0 = b * N + NT + c * RPW
        pltpu.sync_copy(wd_hbm, wd_v)

        def chunk_body(g, carry):
            bs, bi = carry
            pltpu.async_copy(
                emb_hbm.at[pl.ds(row0 + g * RC, RC)], buf, sem).wait()

            def pair_body(p, carry2):
                bs2, bi2 = carry2
                acc0 = jnp.zeros((16,), jnp.float32)
                acc1 = jnp.zeros((16,), jnp.float32)
                for j in range(D // 16):
                    wv = wd_v[pl.ds(j * 16, 16)]
                    acc0 = acc0 + buf[2 * p, pl.ds(j * 16, 16)] * wv
                    acc1 = acc1 + buf[2 * p + 1, pl.ds(j * 16, 16)] * wv
                s0 = jnp.sum(acc0)
                s1 = jnp.sum(acc1)
                i0 = NT + c * RPW + g * RC + 2 * p
                take0 = s0 > bs2
                bs2 = jnp.where(take0, s0, bs2)
                bi2 = jnp.where(take0, i0, bi2)
                take1 = s1 > bs2
                bs2 = jnp.where(take1, s1, bs2)
                bi2 = jnp.where(take1, i0 + 1, bi2)
                return (bs2, bi2)

            return lax.fori_loop(0, RC // 2, pair_body, (bs, bi))

        best, besti = lax.fori_loop(
            0, NCHUNK, chunk_body,
            (jnp.float32(_NEG), jnp.int32(_BIG)))
        bv_v[...] = jnp.full((16,), best, dtype=jnp.float32)
        bi_v[...] = jnp.full((16,), besti, dtype=jnp.int32)
        pltpu.sync_copy(bv_v, out_v.at[wid])
        pltpu.sync_copy(bi_v, out_i.at[wid])

    return sc_score


def _tc_merge_body(tcv_ref, tci_ref, scv_ref, sci_ref, emb_ref,
                   out_emb_ref, out_idx_ref, sem):
    for b in range(B):
        tv = tcv_ref[b]
        ti = tci_ref[b]
        sv = scv_ref[pl.ds(b * WPB, WPB), :]
        si = sci_ref[pl.ds(b * WPB, WPB), :]
        m = jnp.maximum(jnp.max(tv), jnp.max(sv))
        gi = jnp.minimum(
            jnp.min(jnp.where(tv == m, ti, _BIG)),
            jnp.min(jnp.where(sv == m, si, _BIG)))
        out_idx_ref[b] = gi
        cp = pltpu.make_async_copy(
            emb_ref.at[pl.ds(b * N + gi, 1), :],
            out_emb_ref.at[pl.ds(b, 1), :],
            sem)
        cp.start()
        cp.wait()


def _tc_merge(tcv, tci, scv, sci, emb_flat):
    return pl.pallas_call(
        _tc_merge_body,
        in_specs=[
            pl.BlockSpec((B, 1, NCAND), lambda: (0, 0, 0)),
            pl.BlockSpec((B, 1, NCAND), lambda: (0, 0, 0)),
            pl.BlockSpec((NW, 16), lambda: (0, 0)),
            pl.BlockSpec((NW, 16), lambda: (0, 0)),
            pl.BlockSpec(memory_space=pl.ANY),
        ],
        out_specs=[
            pl.BlockSpec((B, D), lambda: (0, 0)),
            pl.BlockSpec(memory_space=pltpu.SMEM),
        ],
        out_shape=[
            jax.ShapeDtypeStruct((B, D), jnp.float32),
            jax.ShapeDtypeStruct((B,), jnp.int32),
        ],
        scratch_shapes=[pltpu.SemaphoreType.DMA],
    )(tcv, tci, scv, sci, emb_flat)


def kernel(token_embeddings, W, b):
    del b
    wd = W[1] - W[0]
    wd2 = wd.reshape(D // 128, 128)
    emb_flat = token_embeddings.reshape(B * N, D)
    scv, sci = _make_sc_score()(emb_flat, wd)
    tcv, tci = _tc_score(token_embeddings, wd2)
    sel_emb, idx = _tc_merge(tcv, tci, scv, sci, emb_flat)
    return (sel_emb, idx)


# speedup vs baseline: 1.1229x; 1.1229x over previous
import functools

import jax
import jax.numpy as jnp
from jax import lax
from jax.experimental import pallas as pl
from jax.experimental.pallas import tpu as pltpu
from jax.experimental.pallas import tpu_sc as plsc

B, N, D = 4, 4096, 2048
NS = 1024
NT = N - NS
BLK = 1024
NBLK = NT // BLK
NCAND = 16
WPB = 8
NW = 32
RPW = NS // WPB
RC = 16
NCHUNK = RPW // RC
_BIG = 2**30
_NEG = float("-inf")


def _tc_score_body(x_ref, wd_ref, val_ref, idx_ref):
    nb = pl.program_id(1)
    x = x_ref[0]
    acc = x[:, 0:128] * wd_ref[0:1, :]
    for k in range(1, D // 128):
        acc = acc + x[:, k * 128:(k + 1) * 128] * wd_ref[k:k + 1, :]
    s = jnp.sum(acc, axis=1, keepdims=True)
    m = jnp.max(s)
    ii = lax.broadcasted_iota(jnp.int32, (BLK, 1), 0)
    am = jnp.min(jnp.where(s == m, ii, _BIG))

    @pl.when(nb == 0)
    def _():
        for k in range(NBLK, NCAND):
            val_ref[0, 0, k] = _NEG
            idx_ref[0, 0, k] = _BIG

    val_ref[0, 0, nb] = m
    idx_ref[0, 0, nb] = am + nb * BLK


def _tc_score(emb, wd2):
    return pl.pallas_call(
        _tc_score_body,
        grid=(B, NBLK),
        compiler_params=pltpu.CompilerParams(
            dimension_semantics=("parallel", "arbitrary")),
        in_specs=[
            pl.BlockSpec((1, BLK, D), lambda b, n: (b, n, 0)),
            pl.BlockSpec((D // 128, 128), lambda b, n: (0, 0)),
        ],
        out_specs=[
            pl.BlockSpec((1, 1, NCAND), lambda b, n: (b, 0, 0),
                         memory_space=pltpu.SMEM),
            pl.BlockSpec((1, 1, NCAND), lambda b, n: (b, 0, 0),
                         memory_space=pltpu.SMEM),
        ],
        out_shape=[
            jax.ShapeDtypeStruct((B, 1, NCAND), jnp.float32),
            jax.ShapeDtypeStruct((B, 1, NCAND), jnp.int32),
        ],
    )(emb, wd2)


@functools.lru_cache(maxsize=1)
def _make_sc_score():
    info = plsc.get_sparse_core_info()
    nc = info.num_cores
    mesh = plsc.VectorSubcoreMesh(core_axis_name="c", subcore_axis_name="s")

    @functools.partial(
        pl.kernel,
        mesh=mesh,
        compiler_params=pltpu.CompilerParams(needs_layout_passes=False),
        out_type=(
            jax.ShapeDtypeStruct((NW, 16), jnp.float32),
            jax.ShapeDtypeStruct((NW, 16), jnp.int32),
        ),
        scratch_types=[
            pltpu.VMEM((D,), jnp.float32),
            pltpu.VMEM((2, RC, D), jnp.float32),
            pltpu.VMEM((16,), jnp.float32),
            pltpu.VMEM((16,), jnp.int32),
            pltpu.SemaphoreType.DMA,
            pltpu.SemaphoreType.DMA,
        ],
    )
    def sc_score(emb_hbm, wd_hbm, out_v, out_i,
                 wd_v, buf, bv_v, bi_v, sem_a, sem_b):
        wid = lax.axis_index("s") * nc + lax.axis_index("c")
        b = wid // WPB
        c = wid % WPB
        row0 = b * N + NT + c * RPW
        pltpu.async_copy(emb_hbm.at[pl.ds(row0, RC)], buf.at[0], sem_a)
        pltpu.sync_copy(wd_hbm, wd_v)

        def compute_chunk(g, buf_h, bs, bi):
            def quad_body(p, carry2):
                bs2, bi2 = carry2
                accs = [jnp.zeros((16,), jnp.float32) for _ in range(4)]
                for j in range(D // 16):
                    wv = wd_v[pl.ds(j * 16, 16)]
                    for r in range(4):
                        accs[r] = accs[r] + (
                            buf_h[4 * p + r, pl.ds(j * 16, 16)] * wv)
                i0 = NT + c * RPW + g * RC + 4 * p
                for r in range(4):
                    sr = jnp.sum(accs[r])
                    take = sr > bs2
                    bs2 = jnp.where(take, sr, bs2)
                    bi2 = jnp.where(take, i0 + r, bi2)
                return (bs2, bi2)

            return lax.fori_loop(0, RC // 4, quad_body, (bs, bi))

        def super_body(i, carry):
            bs, bi = carry
            for half, (sem_w, sem_n) in ((0, (sem_a, sem_b)),
                                         (1, (sem_b, sem_a))):
                g = 2 * i + half
                pltpu.make_async_copy(
                    emb_hbm.at[pl.ds(row0, RC)], buf.at[half], sem_w).wait()

                @pl.when(g + 1 < NCHUNK)
                def _():
                    pltpu.async_copy(
                        emb_hbm.at[pl.ds(row0 + (g + 1) * RC, RC)],
                        buf.at[1 - half], sem_n)

                bs, bi = compute_chunk(g, buf.at[half], bs, bi)
            return (bs, bi)

        best, besti = lax.fori_loop(
            0, NCHUNK // 2, super_body,
            (jnp.float32(_NEG), jnp.int32(_BIG)))
        bv_v[...] = jnp.full((16,), best, dtype=jnp.float32)
        bi_v[...] = jnp.full((16,), besti, dtype=jnp.int32)
        pltpu.sync_copy(bv_v, out_v.at[wid])
        pltpu.sync_copy(bi_v, out_i.at[wid])

    return sc_score


def _tc_merge_body(tcv_ref, tci_ref, scv_ref, sci_ref, emb_ref,
                   out_emb_ref, out_idx_ref, sem):
    for b in range(B):
        tv = tcv_ref[b]
        ti = tci_ref[b]
        sv = scv_ref[pl.ds(b * WPB, WPB), :]
        si = sci_ref[pl.ds(b * WPB, WPB), :]
        m = jnp.maximum(jnp.max(tv), jnp.max(sv))
        gi = jnp.minimum(
            jnp.min(jnp.where(tv == m, ti, _BIG)),
            jnp.min(jnp.where(sv == m, si, _BIG)))
        out_idx_ref[b] = gi
        cp = pltpu.make_async_copy(
            emb_ref.at[pl.ds(b * N + gi, 1), :],
            out_emb_ref.at[pl.ds(b, 1), :],
            sem)
        cp.start()
        cp.wait()


def _tc_merge(tcv, tci, scv, sci, emb_flat):
    return pl.pallas_call(
        _tc_merge_body,
        in_specs=[
            pl.BlockSpec((B, 1, NCAND), lambda: (0, 0, 0)),
            pl.BlockSpec((B, 1, NCAND), lambda: (0, 0, 0)),
            pl.BlockSpec((NW, 16), lambda: (0, 0)),
            pl.BlockSpec((NW, 16), lambda: (0, 0)),
            pl.BlockSpec(memory_space=pl.ANY),
        ],
        out_specs=[
            pl.BlockSpec((B, D), lambda: (0, 0)),
            pl.BlockSpec(memory_space=pltpu.SMEM),
        ],
        out_shape=[
            jax.ShapeDtypeStruct((B, D), jnp.float32),
            jax.ShapeDtypeStruct((B,), jnp.int32),
        ],
        scratch_shapes=[pltpu.SemaphoreType.DMA],
    )(tcv, tci, scv, sci, emb_flat)


def kernel(token_embeddings, W, b):
    del b
    wd = W[1] - W[0]
    wd2 = wd.reshape(D // 128, 128)
    emb_flat = token_embeddings.reshape(B * N, D)
    scv, sci = _make_sc_score()(emb_flat, wd)
    tcv, tci = _tc_score(token_embeddings, wd2)
    sel_emb, idx = _tc_merge(tcv, tci, scv, sci, emb_flat)
    return (sel_emb, idx)
